# baseline (speedup 1.0000x reference)
import functools
import math

import jax
import jax.numpy as jnp
from jax import lax
from jax.experimental import pallas as pl
from jax.experimental.pallas import tpu as pltpu
from jax.experimental.pallas import tpu_sc as plsc

_B = 2
_N = 4096
_K = 16
_D = 128
_RB1 = 256
_RB2 = 128
_GC = 128


def _knn_body(xyz_all_ref, xyz_blk_ref, feat_blk_ref, fc1_w_ref, fc1_b_ref,
              fd1_w_ref, idx_ref, x_ref, p_ref):
    b = pl.program_id(0)
    xyz_all = xyz_all_ref[0]
    xyz_blk = xyz_blk_ref[0]
    dot = lax.dot_general(xyz_blk, xyz_all, (((0,), (0,)), ((), ())),
                          preferred_element_type=jnp.float32)
    sq_c = jnp.sum(xyz_all * xyz_all, axis=0, keepdims=True)
    s = sq_c - 2.0 * dot
    _RG = 32
    _NT = _N // 128
    lane16 = lax.broadcasted_iota(jnp.int32, (_RG, _K), 1).astype(jnp.float32)
    lane_f = lax.broadcasted_iota(jnp.int32, (_RG, 128), 1).astype(jnp.float32)
    INF = jnp.float32(jnp.inf)
    BIGC = jnp.float32(1e9)
    accs, bads = [], []
    for g in range(_RB1 // _RG):
        sg = lax.slice(s, (g * _RG, 0), ((g + 1) * _RG, _N))
        m1 = jnp.full((_RG, 128), INF)
        m2, m3, m4 = m1, m1, m1
        c1 = jnp.full((_RG, 128), BIGC)
        c2, c3, c4 = c1, c1, c1
        for t in range(_NT):
            v = lax.slice(sg, (0, t * 128), (_RG, (t + 1) * 128))
            colv = lane_f + jnp.float32(t * 128)
            l1 = v < m1
            l2 = v < m2
            l3 = v < m3
            l4 = v < m4
            m4 = jnp.where(l4, jnp.where(l3, m3, v), m4)
            c4 = jnp.where(l4, jnp.where(l3, c3, colv), c4)
            m3 = jnp.where(l3, jnp.where(l2, m2, v), m3)
            c3 = jnp.where(l3, jnp.where(l2, c2, colv), c3)
            m2 = jnp.where(l2, jnp.where(l1, m1, v), m2)
            c2 = jnp.where(l2, jnp.where(l1, c1, colv), c2)
            m1 = jnp.where(l1, v, m1)
            c1 = jnp.where(l1, colv, c1)
        acc = jnp.zeros((_RG, _K), jnp.float32)
        m16 = None
        for j in range(_K):
            mv = jnp.min(m1, axis=1, keepdims=True)
            candc = jnp.where(m1 == mv, c1, BIGC)
            idxm = jnp.min(candc, axis=1, keepdims=True)
            acc = jnp.where(lane16 == j, idxm, acc)
            hit = (m1 == mv) & (c1 == idxm)
            m1 = jnp.where(hit, m2, m1)
            c1 = jnp.where(hit, c2, c1)
            m2 = jnp.where(hit, m3, m2)
            c2 = jnp.where(hit, c3, c2)
            m3 = jnp.where(hit, m4, m3)
            c3 = jnp.where(hit, c4, c3)
            m4 = jnp.where(hit, INF, m4)
            m16 = mv
        cnt = jnp.zeros((_RG, 128), jnp.float32)
        for t in range(_NT):
            v = lax.slice(sg, (0, t * 128), (_RG, (t + 1) * 128))
            cnt = cnt + jnp.where(v <= m16, 1.0, 0.0)
        accs.append(acc)
        bads.append(jnp.max(cnt))
    acc_fast = jnp.concatenate(accs, axis=0)
    bad = jnp.max(jnp.stack(bads)) > 4.5

    def _full_extract():
        col = lax.broadcasted_iota(jnp.int32, (_RB1, _N), 1).astype(
            jnp.float32)
        lane = lax.broadcasted_iota(jnp.int32, (_RB1, _K), 1).astype(
            jnp.float32)
        a = jnp.zeros((_RB1, _K), jnp.float32)
        ss = s
        for j in range(_K):
            m = jnp.min(ss, axis=1, keepdims=True)
            cand = jnp.where(ss == m, col, BIGC)
            idxm = jnp.min(cand, axis=1, keepdims=True)
            a = jnp.where(lane == j, idxm, a)
            ss = jnp.where(col == idxm, INF, ss)
        return a

    acc_final = lax.cond(bad, _full_extract, lambda: acc_fast)
    idx_ref[0] = acc_final.astype(jnp.int32) + b * _N
    feat_blk = feat_blk_ref[0]
    x = lax.dot_general(feat_blk, fc1_w_ref[...], (((0,), (0,)), ((), ())),
                        preferred_element_type=jnp.float32)
    x_ref[0] = x + fc1_b_ref[...]
    p_ref[0] = lax.dot_general(xyz_blk, fd1_w_ref[...],
                               (((0,), (0,)), ((), ())),
                               preferred_element_type=jnp.float32)


def _run_knn(xyz, features, fc1_w, fc1_b, fd1_w):
    nb = xyz.shape[0]
    nblk = _N // _RB1
    out_shapes = (
        jax.ShapeDtypeStruct((nb, _N, _K), jnp.int32),
        jax.ShapeDtypeStruct((nb, _N, _D), jnp.float32),
        jax.ShapeDtypeStruct((nb, _N, _D), jnp.float32),
    )
    grid = (nb, nblk)
    return pl.pallas_call(
        _knn_body,
        grid=grid,
        in_specs=[
            pl.BlockSpec((1, 3, _N), lambda b, i: (b, 0, 0)),
            pl.BlockSpec((1, 3, _RB1), lambda b, i: (b, 0, i)),
            pl.BlockSpec((1, 3, _RB1), lambda b, i: (b, 0, i)),
            pl.BlockSpec((3, _D), lambda b, i: (0, 0)),
            pl.BlockSpec((1, _D), lambda b, i: (0, 0)),
            pl.BlockSpec((3, _D), lambda b, i: (0, 0)),
        ],
        out_specs=(
            pl.BlockSpec((1, _RB1, _K), lambda b, i: (b, i, 0)),
            pl.BlockSpec((1, _RB1, _D), lambda b, i: (b, i, 0)),
            pl.BlockSpec((1, _RB1, _D), lambda b, i: (b, i, 0)),
        ),
        out_shape=out_shapes,
    )(xyz, xyz, features, fc1_w, fc1_b.reshape(1, _D), fd1_w)


def _sc_gather(x_tab, p_tab, idx2):
    info = plsc.get_sparse_core_info()
    nc, ns = info.num_cores, info.num_subcores
    nw = nc * ns
    total = idx2.shape[0] * idx2.shape[1]
    per_w = total // nw
    chunks = per_w // _GC

    mesh = plsc.VectorSubcoreMesh(core_axis_name="c", subcore_axis_name="s")

    @functools.partial(
        pl.kernel,
        mesh=mesh,
        out_type=[
            jax.ShapeDtypeStruct((total, _D), jnp.float32),
            jax.ShapeDtypeStruct((total, _D), jnp.float32),
        ],
        scratch_types=[
            pltpu.VMEM((chunks, _GC), jnp.int32),
            pltpu.VMEM((_GC, _D), jnp.float32),
            pltpu.VMEM((_GC, _D), jnp.float32),
            pltpu.VMEM((_GC, _D), jnp.float32),
            pltpu.VMEM((_GC, _D), jnp.float32),
            pltpu.SemaphoreType.DMA,
            pltpu.SemaphoreType.DMA,
            pltpu.SemaphoreType.DMA,
            pltpu.SemaphoreType.DMA,
        ],
    )
    def k(x_hbm, p_hbm, idx_hbm, xg_hbm, pg_hbm, idx_v,
          xb0, pb0, xb1, pb1, sx0, sp0, sx1, sp1):
        wid = lax.axis_index("s") * nc + lax.axis_index("c")
        pltpu.sync_copy(idx_hbm.at[pl.ds(wid * chunks, chunks)], idx_v)
        bufs = ((xb0, pb0, sx0, sp0), (xb1, pb1, sx1, sp1))

        def fire(c, buf):
            xb, pb, sx, sp = buf
            pltpu.async_copy(x_hbm.at[idx_v.at[c]], xb, sx)
            pltpu.async_copy(p_hbm.at[idx_v.at[c]], pb, sp)

        def drain_store(c, buf):
            xb, pb, sx, sp = buf
            pltpu.make_async_copy(x_hbm.at[idx_v.at[c]], xb, sx).wait()
            pltpu.make_async_copy(p_hbm.at[idx_v.at[c]], pb, sp).wait()
            base = wid * per_w + c * _GC
            pltpu.sync_copy(xb, xg_hbm.at[pl.ds(base, _GC)])
            pltpu.sync_copy(pb, pg_hbm.at[pl.ds(base, _GC)])

        fire(0, bufs[0])

        def body(c2, carry):
            ca = c2 * 2
            fire(ca + 1, bufs[1])
            drain_store(ca, bufs[0])

            @pl.when(ca + 2 < chunks)
            def _():
                fire(ca + 2, bufs[0])

            drain_store(ca + 1, bufs[1])
            return carry

        lax.fori_loop(0, chunks // 2, body, 0)

    return k(x_tab, p_tab, idx2)


def _attn_body(xg_ref, pg_ref, xo_ref, po_ref, feat_ref,
               wq_ref, wk_ref, wv_ref, fd1b_ref, fd2_ref, fd2b_ref,
               fg1_ref, fg1b_ref, fg2_ref, fg2b_ref, fc2_ref, fc2b_ref,
               out_ref):
    def mm(a, w):
        return lax.dot_general(a, w, (((1,), (0,)), ((), ())),
                               preferred_element_type=jnp.float32)

    def rep(a):
        return jnp.broadcast_to(a[:, None, :], (_RB2, _K, _D)).reshape(
            _RB2 * _K, _D)

    xo = xo_ref[...]
    q = mm(xo, wq_ref[...])
    xg = xg_ref[...]
    kk = mm(xg, wk_ref[...])
    vv = mm(xg, wv_ref[...])
    pos1 = jnp.maximum(rep(po_ref[...]) - pg_ref[...] + fd1b_ref[...], 0.0)
    pos = mm(pos1, fd2_ref[...]) + fd2b_ref[...]
    g = rep(q) - kk + pos
    h = jnp.maximum(mm(g, fg1_ref[...]) + fg1b_ref[...], 0.0)
    h = mm(h, fg2_ref[...]) + fg2b_ref[...]
    h3 = (h * (1.0 / math.sqrt(_D))).reshape(_RB2, _K, _D)
    mx = jnp.max(h3, axis=1, keepdims=True)
    e = jnp.exp(h3 - mx)
    w = e / jnp.sum(e, axis=1, keepdims=True)
    vp = (vv + pos).reshape(_RB2, _K, _D)
    res = jnp.sum(w * vp, axis=1)
    out_ref[...] = mm(res, fc2_ref[...]) + fc2b_ref[...] + feat_ref[...]


def _run_attn(xg, pg, x_tab, p_tab, feat_nt, wq_w, wk_w, wv_w,
              fd1_b, fd2_w, fd2_b, fg1_w, fg1_b, fg2_w, fg2_b,
              fc2_w, fc2_b):
    npts = x_tab.shape[0]
    nblk = npts // _RB2
    gb = _RB2 * _K
    full = lambda i: (0, 0)
    return pl.pallas_call(
        _attn_body,
        grid=(nblk,),
        in_specs=[
            pl.BlockSpec((gb, _D), lambda i: (i, 0)),
            pl.BlockSpec((gb, _D), lambda i: (i, 0)),
            pl.BlockSpec((_RB2, _D), lambda i: (i, 0)),
            pl.BlockSpec((_RB2, _D), lambda i: (i, 0)),
            pl.BlockSpec((_RB2, 3), lambda i: (i, 0)),
            pl.BlockSpec((_D, _D), full),
            pl.BlockSpec((_D, _D), full),
            pl.BlockSpec((_D, _D), full),
            pl.BlockSpec((1, _D), full),
            pl.BlockSpec((_D, _D), full),
            pl.BlockSpec((1, _D), full),
            pl.BlockSpec((_D, _D), full),
            pl.BlockSpec((1, _D), full),
            pl.BlockSpec((_D, _D), full),
            pl.BlockSpec((1, _D), full),
            pl.BlockSpec((_D, 3), full),
            pl.BlockSpec((1, 3), full),
        ],
        out_specs=pl.BlockSpec((_RB2, 3), lambda i: (i, 0)),
        out_shape=jax.ShapeDtypeStruct((npts, 3), jnp.float32),
    )(xg, pg, x_tab, p_tab, feat_nt,
      wq_w, wk_w, wv_w, fd1_b.reshape(1, _D), fd2_w,
      fd2_b.reshape(1, _D), fg1_w, fg1_b.reshape(1, _D), fg2_w,
      fg2_b.reshape(1, _D), fc2_w, fc2_b.reshape(1, 3))


def kernel(xyz, features, fc1_w, fc1_b, fc2_w, fc2_b, fd1_w, fd1_b,
           fd2_w, fd2_b, fg1_w, fg1_b, fg2_w, fg2_b, wq_w, wk_w, wv_w):
    feat_nt = jnp.transpose(features, (0, 2, 1))
    tabs, gathered = [], []
    for b in range(_B):
        idx, x_tab, p_tab = _run_knn(xyz[b:b + 1], features[b:b + 1],
                                     fc1_w, fc1_b, fd1_w)
        idx2 = idx.reshape((_N * _K) // _GC, _GC)
        x_tab = x_tab.reshape(_N, _D)
        p_tab = p_tab.reshape(_N, _D)
        tabs.append((x_tab, p_tab))
        gathered.append(_sc_gather(x_tab, p_tab, idx2))
    outs = []
    for b in range(_B):
        xg, pg = gathered[b]
        outs.append(_run_attn(xg, pg, tabs[b][0], tabs[b][1],
                              feat_nt[b], wq_w, wk_w, wv_w,
                              fd1_b, fd2_w, fd2_b, fg1_w, fg1_b,
                              fg2_w, fg2_b, fc2_w, fc2_b))
    out = jnp.stack(outs, axis=0)
    return jnp.transpose(out, (0, 2, 1))


# speedup vs baseline: 1.0284x; 1.0284x over previous
import functools
import math

import jax
import jax.numpy as jnp
from jax import lax
from jax.experimental import pallas as pl
from jax.experimental.pallas import tpu as pltpu
from jax.experimental.pallas import tpu_sc as plsc

_B = 2
_N = 4096
_K = 16
_D = 128
_RB1 = 256
_RB2 = 256
_GC = 128


def _knn_body(xyz_all_ref, xyz_blk_ref, feat_blk_ref, fc1_w_ref, fc1_b_ref,
              fd1_w_ref, idx_ref, x_ref, p_ref):
    b = pl.program_id(0)
    xyz_all = xyz_all_ref[0]
    xyz_blk = xyz_blk_ref[0]
    dot = lax.dot_general(xyz_blk, xyz_all, (((0,), (0,)), ((), ())),
                          preferred_element_type=jnp.float32)
    sq_c = jnp.sum(xyz_all * xyz_all, axis=0, keepdims=True)
    s = sq_c - 2.0 * dot
    _RG = 32
    _NT = _N // 128
    lane16 = lax.broadcasted_iota(jnp.int32, (_RG, _K), 1).astype(jnp.float32)
    lane_f = lax.broadcasted_iota(jnp.int32, (_RG, 128), 1).astype(jnp.float32)
    INF = jnp.float32(jnp.inf)
    BIGC = jnp.float32(1e9)
    accs, bads = [], []
    for g in range(_RB1 // _RG):
        sg = lax.slice(s, (g * _RG, 0), ((g + 1) * _RG, _N))
        m1 = jnp.full((_RG, 128), INF)
        m2, m3, m4 = m1, m1, m1
        c1 = jnp.full((_RG, 128), BIGC)
        c2, c3, c4 = c1, c1, c1
        for t in range(_NT):
            v = lax.slice(sg, (0, t * 128), (_RG, (t + 1) * 128))
            colv = lane_f + jnp.float32(t * 128)
            l1 = v < m1
            l2 = v < m2
            l3 = v < m3
            l4 = v < m4
            m4 = jnp.where(l4, jnp.where(l3, m3, v), m4)
            c4 = jnp.where(l4, jnp.where(l3, c3, colv), c4)
            m3 = jnp.where(l3, jnp.where(l2, m2, v), m3)
            c3 = jnp.where(l3, jnp.where(l2, c2, colv), c3)
            m2 = jnp.where(l2, jnp.where(l1, m1, v), m2)
            c2 = jnp.where(l2, jnp.where(l1, c1, colv), c2)
            m1 = jnp.where(l1, v, m1)
            c1 = jnp.where(l1, colv, c1)
        acc = jnp.zeros((_RG, _K), jnp.float32)
        m16 = None
        for j in range(_K):
            mv = jnp.min(m1, axis=1, keepdims=True)
            candc = jnp.where(m1 == mv, c1, BIGC)
            idxm = jnp.min(candc, axis=1, keepdims=True)
            acc = jnp.where(lane16 == j, idxm, acc)
            hit = (m1 == mv) & (c1 == idxm)
            m1 = jnp.where(hit, m2, m1)
            c1 = jnp.where(hit, c2, c1)
            m2 = jnp.where(hit, m3, m2)
            c2 = jnp.where(hit, c3, c2)
            m3 = jnp.where(hit, m4, m3)
            c3 = jnp.where(hit, c4, c3)
            m4 = jnp.where(hit, INF, m4)
            m16 = mv
        cnt = jnp.zeros((_RG, 128), jnp.float32)
        for t in range(_NT):
            v = lax.slice(sg, (0, t * 128), (_RG, (t + 1) * 128))
            cnt = cnt + jnp.where(v <= m16, 1.0, 0.0)
        accs.append(acc)
        bads.append(jnp.max(cnt))
    acc_fast = jnp.concatenate(accs, axis=0)
    bad = jnp.max(jnp.stack(bads)) > 4.5

    def _full_extract():
        col = lax.broadcasted_iota(jnp.int32, (_RB1, _N), 1).astype(
            jnp.float32)
        lane = lax.broadcasted_iota(jnp.int32, (_RB1, _K), 1).astype(
            jnp.float32)
        a = jnp.zeros((_RB1, _K), jnp.float32)
        ss = s
        for j in range(_K):
            m = jnp.min(ss, axis=1, keepdims=True)
            cand = jnp.where(ss == m, col, BIGC)
            idxm = jnp.min(cand, axis=1, keepdims=True)
            a = jnp.where(lane == j, idxm, a)
            ss = jnp.where(col == idxm, INF, ss)
        return a

    acc_final = lax.cond(bad, _full_extract, lambda: acc_fast)
    idx_ref[0] = acc_final.astype(jnp.int32) + b * _N
    feat_blk = feat_blk_ref[0]
    x = lax.dot_general(feat_blk, fc1_w_ref[...], (((0,), (0,)), ((), ())),
                        preferred_element_type=jnp.float32)
    x_ref[0] = x + fc1_b_ref[...]
    p_ref[0] = lax.dot_general(xyz_blk, fd1_w_ref[...],
                               (((0,), (0,)), ((), ())),
                               preferred_element_type=jnp.float32)


def _run_knn(xyz, features, fc1_w, fc1_b, fd1_w):
    nb = xyz.shape[0]
    nblk = _N // _RB1
    out_shapes = (
        jax.ShapeDtypeStruct((nb, _N, _K), jnp.int32),
        jax.ShapeDtypeStruct((nb, _N, _D), jnp.float32),
        jax.ShapeDtypeStruct((nb, _N, _D), jnp.float32),
    )
    grid = (nb, nblk)
    return pl.pallas_call(
        _knn_body,
        grid=grid,
        in_specs=[
            pl.BlockSpec((1, 3, _N), lambda b, i: (b, 0, 0)),
            pl.BlockSpec((1, 3, _RB1), lambda b, i: (b, 0, i)),
            pl.BlockSpec((1, 3, _RB1), lambda b, i: (b, 0, i)),
            pl.BlockSpec((3, _D), lambda b, i: (0, 0)),
            pl.BlockSpec((1, _D), lambda b, i: (0, 0)),
            pl.BlockSpec((3, _D), lambda b, i: (0, 0)),
        ],
        out_specs=(
            pl.BlockSpec((1, _RB1, _K), lambda b, i: (b, i, 0)),
            pl.BlockSpec((1, _RB1, _D), lambda b, i: (b, i, 0)),
            pl.BlockSpec((1, _RB1, _D), lambda b, i: (b, i, 0)),
        ),
        out_shape=out_shapes,
    )(xyz, xyz, features, fc1_w, fc1_b.reshape(1, _D), fd1_w)


def _sc_gather(x_tab, p_tab, idx2):
    info = plsc.get_sparse_core_info()
    nc, ns = info.num_cores, info.num_subcores
    nw = nc * ns
    total = idx2.shape[0] * idx2.shape[1]
    per_w = total // nw
    chunks = per_w // _GC

    mesh = plsc.VectorSubcoreMesh(core_axis_name="c", subcore_axis_name="s")

    @functools.partial(
        pl.kernel,
        mesh=mesh,
        out_type=[
            jax.ShapeDtypeStruct((total, _D), jnp.float32),
            jax.ShapeDtypeStruct((total, _D), jnp.float32),
        ],
        scratch_types=[
            pltpu.VMEM((chunks, _GC), jnp.int32),
            pltpu.VMEM((_GC, _D), jnp.float32),
            pltpu.VMEM((_GC, _D), jnp.float32),
            pltpu.VMEM((_GC, _D), jnp.float32),
            pltpu.VMEM((_GC, _D), jnp.float32),
            pltpu.SemaphoreType.DMA,
            pltpu.SemaphoreType.DMA,
            pltpu.SemaphoreType.DMA,
            pltpu.SemaphoreType.DMA,
        ],
    )
    def k(x_hbm, p_hbm, idx_hbm, xg_hbm, pg_hbm, idx_v,
          xb0, pb0, xb1, pb1, sx0, sp0, sx1, sp1):
        wid = lax.axis_index("s") * nc + lax.axis_index("c")
        pltpu.sync_copy(idx_hbm.at[pl.ds(wid * chunks, chunks)], idx_v)
        bufs = ((xb0, pb0, sx0, sp0), (xb1, pb1, sx1, sp1))

        def fire(c, buf):
            xb, pb, sx, sp = buf
            pltpu.async_copy(x_hbm.at[idx_v.at[c]], xb, sx)
            pltpu.async_copy(p_hbm.at[idx_v.at[c]], pb, sp)

        def drain_store(c, buf):
            xb, pb, sx, sp = buf
            pltpu.make_async_copy(x_hbm.at[idx_v.at[c]], xb, sx).wait()
            pltpu.make_async_copy(p_hbm.at[idx_v.at[c]], pb, sp).wait()
            base = wid * per_w + c * _GC
            pltpu.sync_copy(xb, xg_hbm.at[pl.ds(base, _GC)])
            pltpu.sync_copy(pb, pg_hbm.at[pl.ds(base, _GC)])

        fire(0, bufs[0])

        def body(c2, carry):
            ca = c2 * 2
            fire(ca + 1, bufs[1])
            drain_store(ca, bufs[0])

            @pl.when(ca + 2 < chunks)
            def _():
                fire(ca + 2, bufs[0])

            drain_store(ca + 1, bufs[1])
            return carry

        lax.fori_loop(0, chunks // 2, body, 0)

    return k(x_tab, p_tab, idx2)


def _attn_body(xg_ref, pg_ref, xo_ref, po_ref, feat_ref,
               wq_ref, wk_ref, wv_ref, fd1b_ref, fd2_ref, fd2b_ref,
               fg1_ref, fg1b_ref, fg2_ref, fg2b_ref, fc2_ref, fc2b_ref,
               out_ref):
    def mm(a, w):
        return lax.dot_general(a, w, (((1,), (0,)), ((), ())),
                               preferred_element_type=jnp.float32)

    def rep(a):
        return jnp.broadcast_to(a[:, None, :], (_RB2, _K, _D)).reshape(
            _RB2 * _K, _D)

    xo = xo_ref[...]
    q = mm(xo, wq_ref[...])
    xg = xg_ref[...]
    kk = mm(xg, wk_ref[...])
    vv = mm(xg, wv_ref[...])
    pos1 = jnp.maximum(rep(po_ref[...]) - pg_ref[...] + fd1b_ref[...], 0.0)
    pos = mm(pos1, fd2_ref[...]) + fd2b_ref[...]
    g = rep(q) - kk + pos
    h = jnp.maximum(mm(g, fg1_ref[...]) + fg1b_ref[...], 0.0)
    h = mm(h, fg2_ref[...]) + fg2b_ref[...]
    h3 = (h * (1.0 / math.sqrt(_D))).reshape(_RB2, _K, _D)
    mx = jnp.max(h3, axis=1, keepdims=True)
    e = jnp.exp(h3 - mx)
    w = e / jnp.sum(e, axis=1, keepdims=True)
    vp = (vv + pos).reshape(_RB2, _K, _D)
    res = jnp.sum(w * vp, axis=1)
    out_ref[...] = mm(res, fc2_ref[...]) + fc2b_ref[...] + feat_ref[...]


def _run_attn(xg, pg, x_tab, p_tab, feat_nt, wq_w, wk_w, wv_w,
              fd1_b, fd2_w, fd2_b, fg1_w, fg1_b, fg2_w, fg2_b,
              fc2_w, fc2_b):
    npts = x_tab.shape[0]
    nblk = npts // _RB2
    gb = _RB2 * _K
    full = lambda i: (0, 0)
    return pl.pallas_call(
        _attn_body,
        grid=(nblk,),
        in_specs=[
            pl.BlockSpec((gb, _D), lambda i: (i, 0)),
            pl.BlockSpec((gb, _D), lambda i: (i, 0)),
            pl.BlockSpec((_RB2, _D), lambda i: (i, 0)),
            pl.BlockSpec((_RB2, _D), lambda i: (i, 0)),
            pl.BlockSpec((_RB2, 3), lambda i: (i, 0)),
            pl.BlockSpec((_D, _D), full),
            pl.BlockSpec((_D, _D), full),
            pl.BlockSpec((_D, _D), full),
            pl.BlockSpec((1, _D), full),
            pl.BlockSpec((_D, _D), full),
            pl.BlockSpec((1, _D), full),
            pl.BlockSpec((_D, _D), full),
            pl.BlockSpec((1, _D), full),
            pl.BlockSpec((_D, _D), full),
            pl.BlockSpec((1, _D), full),
            pl.BlockSpec((_D, 3), full),
            pl.BlockSpec((1, 3), full),
        ],
        out_specs=pl.BlockSpec((_RB2, 3), lambda i: (i, 0)),
        out_shape=jax.ShapeDtypeStruct((npts, 3), jnp.float32),
    )(xg, pg, x_tab, p_tab, feat_nt,
      wq_w, wk_w, wv_w, fd1_b.reshape(1, _D), fd2_w,
      fd2_b.reshape(1, _D), fg1_w, fg1_b.reshape(1, _D), fg2_w,
      fg2_b.reshape(1, _D), fc2_w, fc2_b.reshape(1, 3))


def kernel(xyz, features, fc1_w, fc1_b, fc2_w, fc2_b, fd1_w, fd1_b,
           fd2_w, fd2_b, fg1_w, fg1_b, fg2_w, fg2_b, wq_w, wk_w, wv_w):
    feat_nt = jnp.transpose(features, (0, 2, 1))
    tabs, gathered = [], []
    for b in range(_B):
        idx, x_tab, p_tab = _run_knn(xyz[b:b + 1], features[b:b + 1],
                                     fc1_w, fc1_b, fd1_w)
        idx2 = idx.reshape((_N * _K) // _GC, _GC)
        x_tab = x_tab.reshape(_N, _D)
        p_tab = p_tab.reshape(_N, _D)
        tabs.append((x_tab, p_tab))
        gathered.append(_sc_gather(x_tab, p_tab, idx2))
    outs = []
    for b in range(_B):
        xg, pg = gathered[b]
        outs.append(_run_attn(xg, pg, tabs[b][0], tabs[b][1],
                              feat_nt[b], wq_w, wk_w, wv_w,
                              fd1_b, fd2_w, fd2_b, fg1_w, fg1_b,
                              fg2_w, fg2_b, fc2_w, fc2_b))
    out = jnp.stack(outs, axis=0)
    return jnp.transpose(out, (0, 2, 1))


# speedup vs baseline: 1.0338x; 1.0053x over previous
import functools
import math

import jax
import jax.numpy as jnp
from jax import lax
from jax.experimental import pallas as pl
from jax.experimental.pallas import tpu as pltpu
from jax.experimental.pallas import tpu_sc as plsc

_B = 2
_N = 4096
_K = 16
_D = 128
_RB1 = 256
_RB2 = 512
_GC = 128


def _knn_body(xyz_all_ref, xyz_blk_ref, feat_blk_ref, fc1_w_ref, fc1_b_ref,
              fd1_w_ref, idx_ref, x_ref, p_ref):
    b = pl.program_id(0)
    xyz_all = xyz_all_ref[0]
    xyz_blk = xyz_blk_ref[0]
    dot = lax.dot_general(xyz_blk, xyz_all, (((0,), (0,)), ((), ())),
                          preferred_element_type=jnp.float32)
    sq_c = jnp.sum(xyz_all * xyz_all, axis=0, keepdims=True)
    s = sq_c - 2.0 * dot
    _RG = 32
    _NT = _N // 128
    lane16 = lax.broadcasted_iota(jnp.int32, (_RG, _K), 1).astype(jnp.float32)
    lane_f = lax.broadcasted_iota(jnp.int32, (_RG, 128), 1).astype(jnp.float32)
    INF = jnp.float32(jnp.inf)
    BIGC = jnp.float32(1e9)
    accs, bads = [], []
    for g in range(_RB1 // _RG):
        sg = lax.slice(s, (g * _RG, 0), ((g + 1) * _RG, _N))
        m1 = jnp.full((_RG, 128), INF)
        m2, m3, m4 = m1, m1, m1
        c1 = jnp.full((_RG, 128), BIGC)
        c2, c3, c4 = c1, c1, c1
        for t in range(_NT):
            v = lax.slice(sg, (0, t * 128), (_RG, (t + 1) * 128))
            colv = lane_f + jnp.float32(t * 128)
            l1 = v < m1
            l2 = v < m2
            l3 = v < m3
            l4 = v < m4
            m4 = jnp.where(l4, jnp.where(l3, m3, v), m4)
            c4 = jnp.where(l4, jnp.where(l3, c3, colv), c4)
            m3 = jnp.where(l3, jnp.where(l2, m2, v), m3)
            c3 = jnp.where(l3, jnp.where(l2, c2, colv), c3)
            m2 = jnp.where(l2, jnp.where(l1, m1, v), m2)
            c2 = jnp.where(l2, jnp.where(l1, c1, colv), c2)
            m1 = jnp.where(l1, v, m1)
            c1 = jnp.where(l1, colv, c1)
        acc = jnp.zeros((_RG, _K), jnp.float32)
        m16 = None
        for j in range(_K):
            mv = jnp.min(m1, axis=1, keepdims=True)
            candc = jnp.where(m1 == mv, c1, BIGC)
            idxm = jnp.min(candc, axis=1, keepdims=True)
            acc = jnp.where(lane16 == j, idxm, acc)
            hit = (m1 == mv) & (c1 == idxm)
            m1 = jnp.where(hit, m2, m1)
            c1 = jnp.where(hit, c2, c1)
            m2 = jnp.where(hit, m3, m2)
            c2 = jnp.where(hit, c3, c2)
            m3 = jnp.where(hit, m4, m3)
            c3 = jnp.where(hit, c4, c3)
            m4 = jnp.where(hit, INF, m4)
            m16 = mv
        cnt = jnp.zeros((_RG, 128), jnp.float32)
        for t in range(_NT):
            v = lax.slice(sg, (0, t * 128), (_RG, (t + 1) * 128))
            cnt = cnt + jnp.where(v <= m16, 1.0, 0.0)
        accs.append(acc)
        bads.append(jnp.max(cnt))
    acc_fast = jnp.concatenate(accs, axis=0)
    bad = jnp.max(jnp.stack(bads)) > 4.5

    def _full_extract():
        col = lax.broadcasted_iota(jnp.int32, (_RB1, _N), 1).astype(
            jnp.float32)
        lane = lax.broadcasted_iota(jnp.int32, (_RB1, _K), 1).astype(
            jnp.float32)
        a = jnp.zeros((_RB1, _K), jnp.float32)
        ss = s
        for j in range(_K):
            m = jnp.min(ss, axis=1, keepdims=True)
            cand = jnp.where(ss == m, col, BIGC)
            idxm = jnp.min(cand, axis=1, keepdims=True)
            a = jnp.where(lane == j, idxm, a)
            ss = jnp.where(col == idxm, INF, ss)
        return a

    acc_final = lax.cond(bad, _full_extract, lambda: acc_fast)
    idx_ref[0] = acc_final.astype(jnp.int32) + b * _N
    feat_blk = feat_blk_ref[0]
    x = lax.dot_general(feat_blk, fc1_w_ref[...], (((0,), (0,)), ((), ())),
                        preferred_element_type=jnp.float32)
    x_ref[0] = x + fc1_b_ref[...]
    p_ref[0] = lax.dot_general(xyz_blk, fd1_w_ref[...],
                               (((0,), (0,)), ((), ())),
                               preferred_element_type=jnp.float32)


def _run_knn(xyz, features, fc1_w, fc1_b, fd1_w):
    nb = xyz.shape[0]
    nblk = _N // _RB1
    out_shapes = (
        jax.ShapeDtypeStruct((nb, _N, _K), jnp.int32),
        jax.ShapeDtypeStruct((nb, _N, _D), jnp.float32),
        jax.ShapeDtypeStruct((nb, _N, _D), jnp.float32),
    )
    grid = (nb, nblk)
    return pl.pallas_call(
        _knn_body,
        grid=grid,
        in_specs=[
            pl.BlockSpec((1, 3, _N), lambda b, i: (b, 0, 0)),
            pl.BlockSpec((1, 3, _RB1), lambda b, i: (b, 0, i)),
            pl.BlockSpec((1, 3, _RB1), lambda b, i: (b, 0, i)),
            pl.BlockSpec((3, _D), lambda b, i: (0, 0)),
            pl.BlockSpec((1, _D), lambda b, i: (0, 0)),
            pl.BlockSpec((3, _D), lambda b, i: (0, 0)),
        ],
        out_specs=(
            pl.BlockSpec((1, _RB1, _K), lambda b, i: (b, i, 0)),
            pl.BlockSpec((1, _RB1, _D), lambda b, i: (b, i, 0)),
            pl.BlockSpec((1, _RB1, _D), lambda b, i: (b, i, 0)),
        ),
        out_shape=out_shapes,
    )(xyz, xyz, features, fc1_w, fc1_b.reshape(1, _D), fd1_w)


def _sc_gather(x_tab, p_tab, idx2):
    info = plsc.get_sparse_core_info()
    nc, ns = info.num_cores, info.num_subcores
    nw = nc * ns
    total = idx2.shape[0] * idx2.shape[1]
    per_w = total // nw
    chunks = per_w // _GC

    mesh = plsc.VectorSubcoreMesh(core_axis_name="c", subcore_axis_name="s")

    @functools.partial(
        pl.kernel,
        mesh=mesh,
        out_type=[
            jax.ShapeDtypeStruct((total, _D), jnp.float32),
            jax.ShapeDtypeStruct((total, _D), jnp.float32),
        ],
        scratch_types=[
            pltpu.VMEM((chunks, _GC), jnp.int32),
            pltpu.VMEM((_GC, _D), jnp.float32),
            pltpu.VMEM((_GC, _D), jnp.float32),
            pltpu.VMEM((_GC, _D), jnp.float32),
            pltpu.VMEM((_GC, _D), jnp.float32),
            pltpu.SemaphoreType.DMA,
            pltpu.SemaphoreType.DMA,
            pltpu.SemaphoreType.DMA,
            pltpu.SemaphoreType.DMA,
        ],
    )
    def k(x_hbm, p_hbm, idx_hbm, xg_hbm, pg_hbm, idx_v,
          xb0, pb0, xb1, pb1, sx0, sp0, sx1, sp1):
        wid = lax.axis_index("s") * nc + lax.axis_index("c")
        pltpu.sync_copy(idx_hbm.at[pl.ds(wid * chunks, chunks)], idx_v)
        bufs = ((xb0, pb0, sx0, sp0), (xb1, pb1, sx1, sp1))

        def fire(c, buf):
            xb, pb, sx, sp = buf
            pltpu.async_copy(x_hbm.at[idx_v.at[c]], xb, sx)
            pltpu.async_copy(p_hbm.at[idx_v.at[c]], pb, sp)

        def drain_store(c, buf):
            xb, pb, sx, sp = buf
            pltpu.make_async_copy(x_hbm.at[idx_v.at[c]], xb, sx).wait()
            pltpu.make_async_copy(p_hbm.at[idx_v.at[c]], pb, sp).wait()
            base = wid * per_w + c * _GC
            pltpu.sync_copy(xb, xg_hbm.at[pl.ds(base, _GC)])
            pltpu.sync_copy(pb, pg_hbm.at[pl.ds(base, _GC)])

        fire(0, bufs[0])

        def body(c2, carry):
            ca = c2 * 2
            fire(ca + 1, bufs[1])
            drain_store(ca, bufs[0])

            @pl.when(ca + 2 < chunks)
            def _():
                fire(ca + 2, bufs[0])

            drain_store(ca + 1, bufs[1])
            return carry

        lax.fori_loop(0, chunks // 2, body, 0)

    return k(x_tab, p_tab, idx2)


def _attn_body(xg_ref, pg_ref, xo_ref, po_ref, feat_ref,
               wq_ref, wk_ref, wv_ref, fd1b_ref, fd2_ref, fd2b_ref,
               fg1_ref, fg1b_ref, fg2_ref, fg2b_ref, fc2_ref, fc2b_ref,
               out_ref):
    def mm(a, w):
        return lax.dot_general(a, w, (((1,), (0,)), ((), ())),
                               preferred_element_type=jnp.float32)

    def rep(a):
        return jnp.broadcast_to(a[:, None, :], (_RB2, _K, _D)).reshape(
            _RB2 * _K, _D)

    xo = xo_ref[...]
    q = mm(xo, wq_ref[...])
    xg = xg_ref[...]
    kk = mm(xg, wk_ref[...])
    vv = mm(xg, wv_ref[...])
    pos1 = jnp.maximum(rep(po_ref[...]) - pg_ref[...] + fd1b_ref[...], 0.0)
    pos = mm(pos1, fd2_ref[...]) + fd2b_ref[...]
    g = rep(q) - kk + pos
    h = jnp.maximum(mm(g, fg1_ref[...]) + fg1b_ref[...], 0.0)
    h = mm(h, fg2_ref[...]) + fg2b_ref[...]
    h3 = (h * (1.0 / math.sqrt(_D))).reshape(_RB2, _K, _D)
    mx = jnp.max(h3, axis=1, keepdims=True)
    e = jnp.exp(h3 - mx)
    w = e / jnp.sum(e, axis=1, keepdims=True)
    vp = (vv + pos).reshape(_RB2, _K, _D)
    res = jnp.sum(w * vp, axis=1)
    out_ref[...] = mm(res, fc2_ref[...]) + fc2b_ref[...] + feat_ref[...]


def _run_attn(xg, pg, x_tab, p_tab, feat_nt, wq_w, wk_w, wv_w,
              fd1_b, fd2_w, fd2_b, fg1_w, fg1_b, fg2_w, fg2_b,
              fc2_w, fc2_b):
    npts = x_tab.shape[0]
    nblk = npts // _RB2
    gb = _RB2 * _K
    full = lambda i: (0, 0)
    return pl.pallas_call(
        _attn_body,
        grid=(nblk,),
        in_specs=[
            pl.BlockSpec((gb, _D), lambda i: (i, 0)),
            pl.BlockSpec((gb, _D), lambda i: (i, 0)),
            pl.BlockSpec((_RB2, _D), lambda i: (i, 0)),
            pl.BlockSpec((_RB2, _D), lambda i: (i, 0)),
            pl.BlockSpec((_RB2, 3), lambda i: (i, 0)),
            pl.BlockSpec((_D, _D), full),
            pl.BlockSpec((_D, _D), full),
            pl.BlockSpec((_D, _D), full),
            pl.BlockSpec((1, _D), full),
            pl.BlockSpec((_D, _D), full),
            pl.BlockSpec((1, _D), full),
            pl.BlockSpec((_D, _D), full),
            pl.BlockSpec((1, _D), full),
            pl.BlockSpec((_D, _D), full),
            pl.BlockSpec((1, _D), full),
            pl.BlockSpec((_D, 3), full),
            pl.BlockSpec((1, 3), full),
        ],
        out_specs=pl.BlockSpec((_RB2, 3), lambda i: (i, 0)),
        out_shape=jax.ShapeDtypeStruct((npts, 3), jnp.float32),
    )(xg, pg, x_tab, p_tab, feat_nt,
      wq_w, wk_w, wv_w, fd1_b.reshape(1, _D), fd2_w,
      fd2_b.reshape(1, _D), fg1_w, fg1_b.reshape(1, _D), fg2_w,
      fg2_b.reshape(1, _D), fc2_w, fc2_b.reshape(1, 3))


def kernel(xyz, features, fc1_w, fc1_b, fc2_w, fc2_b, fd1_w, fd1_b,
           fd2_w, fd2_b, fg1_w, fg1_b, fg2_w, fg2_b, wq_w, wk_w, wv_w):
    feat_nt = jnp.transpose(features, (0, 2, 1))
    tabs, gathered = [], []
    for b in range(_B):
        idx, x_tab, p_tab = _run_knn(xyz[b:b + 1], features[b:b + 1],
                                     fc1_w, fc1_b, fd1_w)
        idx2 = idx.reshape((_N * _K) // _GC, _GC)
        x_tab = x_tab.reshape(_N, _D)
        p_tab = p_tab.reshape(_N, _D)
        tabs.append((x_tab, p_tab))
        gathered.append(_sc_gather(x_tab, p_tab, idx2))
    outs = []
    for b in range(_B):
        xg, pg = gathered[b]
        outs.append(_run_attn(xg, pg, tabs[b][0], tabs[b][1],
                              feat_nt[b], wq_w, wk_w, wv_w,
                              fd1_b, fd2_w, fd2_b, fg1_w, fg1_b,
                              fg2_w, fg2_b, fc2_w, fc2_b))
    out = jnp.stack(outs, axis=0)
    return jnp.transpose(out, (0, 2, 1))
